# baseline (speedup 1.0000x reference)
import jax
import jax.numpy as jnp
from jax import lax
from jax.experimental import pallas as pl
from jax.experimental.pallas import tpu as pltpu
from jax.experimental.pallas import tpu_sc as plsc

_B = 16384
_F = 64
_NW = 32
_BPW = _B // _NW
_CHUNK = 128
_NCHUNK = _BPW // _CHUNK
_GROUPS = _BPW // 16


def _clmf_body(uidx_hbm, iidx_hbm, utab_hbm, itab_hbm, wb_hbm, out_hbm,
               uidx_v, iidx_v, urows_v, irows_v, wb_v, out_v, sem):
    nc = 2
    wid = lax.axis_index("s") * nc + lax.axis_index("c")

    pltpu.sync_copy(uidx_hbm.at[pl.ds(wid * _NCHUNK, _NCHUNK)], uidx_v)
    pltpu.sync_copy(iidx_hbm.at[pl.ds(wid * _NCHUNK, _NCHUNK)], iidx_v)
    pltpu.sync_copy(wb_hbm, wb_v)

    descs = []
    for j in range(_NCHUNK):
        descs.append(pltpu.async_copy(
            utab_hbm.at[uidx_v.at[j]],
            urows_v.at[pl.ds(j * _CHUNK, _CHUNK)], sem))
        descs.append(pltpu.async_copy(
            itab_hbm.at[iidx_v.at[j]],
            irows_v.at[pl.ds(j * _CHUNK, _CHUNK)], sem))
    for d in descs:
        d.wait()

    bvec = wb_v[pl.ds(_F, 16)]
    wvecs = [wb_v[pl.ds(c * 16, 16)] for c in range(_F // 16)]
    lane = lax.iota(jnp.int32, 16)

    def group_body(g, carry):
        rowbase = g * 16
        rows = rowbase + lane
        acc = bvec
        for f in range(_F):
            wf = wvecs[f // 16][f % 16]
            colf = jnp.full((16,), f, jnp.int32)
            u = plsc.load_gather(urows_v, [rows, colf])
            iv = plsc.load_gather(irows_v, [rows, colf])
            acc = acc + u * iv * wf
        out_v[pl.ds(rowbase, 16)] = acc
        return carry

    lax.fori_loop(0, _GROUPS, group_body, 0)

    pltpu.sync_copy(out_v, out_hbm.at[pl.ds(wid * _BPW, _BPW)])


def kernel(user, item, embed_user_w, embed_item_w, predict_w, predict_b):
    uidx = user.astype(jnp.int32).reshape(_B // _CHUNK, _CHUNK)
    iidx = item.astype(jnp.int32).reshape(_B // _CHUNK, _CHUNK)
    wb = jnp.concatenate([predict_w.reshape(_F).astype(jnp.float32),
                          jnp.broadcast_to(predict_b.astype(jnp.float32), (16,))])

    mesh = plsc.VectorSubcoreMesh(core_axis_name="c", subcore_axis_name="s")
    run = pl.kernel(
        _clmf_body,
        out_type=jax.ShapeDtypeStruct((_B,), jnp.float32),
        mesh=mesh,
        compiler_params=pltpu.CompilerParams(needs_layout_passes=False,
                                             use_tc_tiling_on_sc=False),
        scratch_types=[
            pltpu.VMEM((_NCHUNK, _CHUNK), jnp.int32),
            pltpu.VMEM((_NCHUNK, _CHUNK), jnp.int32),
            pltpu.VMEM((_BPW, _F), jnp.float32),
            pltpu.VMEM((_BPW, _F), jnp.float32),
            pltpu.VMEM((_F + 16,), jnp.float32),
            pltpu.VMEM((_BPW,), jnp.float32),
            pltpu.SemaphoreType.DMA,
        ],
    )
    return run(uidx, iidx, embed_user_w, embed_item_w, wb)
